# baseline (speedup 1.0000x reference)
import functools

import jax
import jax.numpy as jnp
from jax import lax
from jax.experimental import pallas as pl
from jax.experimental.pallas import tpu as pltpu
from jax.experimental.pallas import tpu_sc as plsc

N_NODES = 10000
N_EDGES = 320000
D = 128

NC = 2
NS = 16
NW = NC * NS

BATCH = 128
CHB = 4
NB = 80
NCHB = NB // CHB
EPT = NB * BATCH
E_PAD = EPT * NW
N_PAD = N_NODES + 16
ROWS_PER_SUB = 624
CHUNK = 104
NCHUNK = ROWS_PER_SUB // CHUNK
REM_BASE = ROWS_PER_SUB * NS
REM_ROWS = N_NODES - REM_BASE


def _sc_aggregate(x, edge_t):
  mesh = plsc.VectorSubcoreMesh(core_axis_name="c", subcore_axis_name="s")

  @functools.partial(
      pl.kernel,
      mesh=mesh,
      out_type=jax.ShapeDtypeStruct((NC, N_NODES, D), jnp.float32),
      scratch_types=[
          pltpu.VMEM((2 * CHB, BATCH), jnp.int32),
          pltpu.VMEM((BATCH, D), jnp.float32),
          pltpu.VMEM((CHUNK, D), jnp.float32),
          pltpu.VMEM_SHARED((N_PAD, D), jnp.float32),
          pltpu.SemaphoreType.DMA,
          pltpu.SemaphoreType.DMA,
      ],
  )
  def k(x_hbm, edge_hbm, out_hbm, eidx, rows, zbuf, aggr, sem, sem2):
    c = lax.axis_index("c")
    s = lax.axis_index("s")
    wid = s * NC + c

    zero = jnp.zeros((16,), jnp.float32)

    def zrow(r, carry):
      for blk in range(D // 16):
        zbuf[r, pl.ds(blk * 16, 16)] = zero
      return carry

    lax.fori_loop(0, CHUNK, zrow, 0)
    r0 = s * ROWS_PER_SUB
    for kk in range(NCHUNK):
      pltpu.sync_copy(zbuf, aggr.at[pl.ds(r0 + kk * CHUNK, CHUNK)])

    @pl.when(s == NS - 1)
    def _zero_rem():
      pltpu.sync_copy(zbuf.at[pl.ds(0, REM_ROWS)],
                      aggr.at[pl.ds(REM_BASE, REM_ROWS)])

    plsc.subcore_barrier()

    def body(ci, carry):
      @pl.when(ci > 0)
      def _wait_prev_scatter():
        pltpu.make_async_copy(rows, aggr.at[eidx.at[1]], sem2).wait()

      pltpu.sync_copy(edge_hbm.at[wid].at[ci], eidx)
      for k in range(CHB):
        if k > 0:
          pltpu.make_async_copy(rows, aggr.at[eidx.at[2 * k - 1]],
                                sem2).wait()
        pltpu.async_copy(x_hbm.at[eidx.at[2 * k]], rows, sem).wait()
        pltpu.async_copy(rows, aggr.at[eidx.at[2 * k + 1]], sem2, add=True)
      return carry

    lax.fori_loop(0, NCHB, body, 0)
    pltpu.make_async_copy(rows, aggr.at[eidx.at[1]], sem2).wait()
    plsc.subcore_barrier()

    for kk in range(NCHUNK):
      pltpu.sync_copy(aggr.at[pl.ds(r0 + kk * CHUNK, CHUNK)], zbuf)
      pltpu.sync_copy(zbuf, out_hbm.at[c].at[pl.ds(r0 + kk * CHUNK, CHUNK)])

    @pl.when(s == NS - 1)
    def _write_rem():
      pltpu.sync_copy(aggr.at[pl.ds(REM_BASE, REM_ROWS)],
                      rows.at[pl.ds(0, REM_ROWS)])
      pltpu.sync_copy(rows.at[pl.ds(0, REM_ROWS)],
                      out_hbm.at[c].at[pl.ds(REM_BASE, REM_ROWS)])

  return k(x, edge_t)


def _mlp_body(relu_out, x_ref, p_ref, wa_ref, ba_ref, wb_ref, bb_ref, o_ref):
  h = x_ref[...] + p_ref[0] + p_ref[1]
  t = jnp.dot(h, wa_ref[...], preferred_element_type=jnp.float32)
  t = jnp.maximum(t + ba_ref[...], 0.0)
  y = jnp.dot(t, wb_ref[...], preferred_element_type=jnp.float32)
  y = y + bb_ref[...]
  if relu_out:
    y = jnp.maximum(y, 0.0)
  o_ref[...] = y


_ROWS = 1000


def _tc_mlp(x, parts, Wa, ba, Wb, bb, relu_out):
  return pl.pallas_call(
      functools.partial(_mlp_body, relu_out),
      grid=(N_NODES // _ROWS,),
      in_specs=[
          pl.BlockSpec((_ROWS, D), lambda i: (i, 0)),
          pl.BlockSpec((NC, _ROWS, D), lambda i: (0, i, 0)),
          pl.BlockSpec((D, D), lambda i: (0, 0)),
          pl.BlockSpec((1, D), lambda i: (0, 0)),
          pl.BlockSpec((D, D), lambda i: (0, 0)),
          pl.BlockSpec((1, D), lambda i: (0, 0)),
      ],
      out_specs=pl.BlockSpec((_ROWS, D), lambda i: (i, 0)),
      out_shape=jax.ShapeDtypeStruct((N_NODES, D), jnp.float32),
  )(x, parts, Wa, ba.reshape(1, D), Wb, bb.reshape(1, D))


def kernel(x, edge_index, W1, b1, W2, b2, W3, b3, W4, b4):
  src = edge_index[0].astype(jnp.int32)
  dst = edge_index[1].astype(jnp.int32)
  pad = E_PAD - N_EDGES
  src_t = jnp.concatenate([src, jnp.zeros((pad,), jnp.int32)]).reshape(
      NW, NB, BATCH)
  dst_t = jnp.concatenate([dst, jnp.full((pad,), N_NODES, jnp.int32)]).reshape(
      NW, NB, BATCH)
  edge_t = jnp.stack([src_t, dst_t], axis=2).reshape(
      NW, NCHB, 2 * CHB, BATCH)

  p1 = _sc_aggregate(x, edge_t)
  h = _tc_mlp(x, p1, W1, b1, W2, b2, relu_out=True)
  p2 = _sc_aggregate(h, edge_t)
  return _tc_mlp(h, p2, W3, b3, W4, b4, relu_out=False)


# speedup vs baseline: 1.0009x; 1.0009x over previous
import functools

import jax
import jax.numpy as jnp
from jax import lax
from jax.experimental import pallas as pl
from jax.experimental.pallas import tpu as pltpu
from jax.experimental.pallas import tpu_sc as plsc

N_NODES = 10000
N_EDGES = 320000
D = 128

NC = 2
NS = 16
NW = NC * NS

BATCH = 128
CHB = 4
NB = 80
NCHB = NB // CHB
EPT = NB * BATCH
E_PAD = EPT * NW
TRASH = 1024
N_PAD = N_NODES + TRASH
ROWS_PER_SUB = 624
CHUNK = 104
NCHUNK = ROWS_PER_SUB // CHUNK
REM_BASE = ROWS_PER_SUB * NS
REM_ROWS = N_NODES - REM_BASE


def _sc_aggregate(x, edge_t):
  mesh = plsc.VectorSubcoreMesh(core_axis_name="c", subcore_axis_name="s")

  @functools.partial(
      pl.kernel,
      mesh=mesh,
      out_type=jax.ShapeDtypeStruct((NC, N_NODES, D), jnp.float32),
      scratch_types=[
          pltpu.VMEM((2 * CHB, BATCH), jnp.int32),
          pltpu.VMEM((BATCH, D), jnp.float32),
          pltpu.VMEM((CHUNK, D), jnp.float32),
          pltpu.VMEM_SHARED((N_PAD, D), jnp.float32),
          pltpu.SemaphoreType.DMA,
          pltpu.SemaphoreType.DMA,
      ],
  )
  def k(x_hbm, edge_hbm, out_hbm, eidx, rows, zbuf, aggr, sem, sem2):
    c = lax.axis_index("c")
    s = lax.axis_index("s")
    wid = s * NC + c

    zero = jnp.zeros((16,), jnp.float32)

    def zrow(r, carry):
      for blk in range(D // 16):
        zbuf[r, pl.ds(blk * 16, 16)] = zero
      return carry

    lax.fori_loop(0, CHUNK, zrow, 0)
    r0 = s * ROWS_PER_SUB
    for kk in range(NCHUNK):
      pltpu.sync_copy(zbuf, aggr.at[pl.ds(r0 + kk * CHUNK, CHUNK)])

    @pl.when(s == NS - 1)
    def _zero_rem():
      pltpu.sync_copy(zbuf.at[pl.ds(0, REM_ROWS)],
                      aggr.at[pl.ds(REM_BASE, REM_ROWS)])

    plsc.subcore_barrier()

    def body(ci, carry):
      @pl.when(ci > 0)
      def _wait_prev_scatter():
        pltpu.make_async_copy(rows, aggr.at[eidx.at[1]], sem2).wait()

      pltpu.sync_copy(edge_hbm.at[wid].at[ci], eidx)
      for k in range(CHB):
        if k > 0:
          pltpu.make_async_copy(rows, aggr.at[eidx.at[2 * k - 1]],
                                sem2).wait()
        pltpu.async_copy(x_hbm.at[eidx.at[2 * k]], rows, sem).wait()
        pltpu.async_copy(rows, aggr.at[eidx.at[2 * k + 1]], sem2, add=True)
      return carry

    lax.fori_loop(0, NCHB, body, 0)
    pltpu.make_async_copy(rows, aggr.at[eidx.at[1]], sem2).wait()
    plsc.subcore_barrier()

    for kk in range(NCHUNK):
      pltpu.sync_copy(aggr.at[pl.ds(r0 + kk * CHUNK, CHUNK)], zbuf)
      pltpu.sync_copy(zbuf, out_hbm.at[c].at[pl.ds(r0 + kk * CHUNK, CHUNK)])

    @pl.when(s == NS - 1)
    def _write_rem():
      pltpu.sync_copy(aggr.at[pl.ds(REM_BASE, REM_ROWS)],
                      rows.at[pl.ds(0, REM_ROWS)])
      pltpu.sync_copy(rows.at[pl.ds(0, REM_ROWS)],
                      out_hbm.at[c].at[pl.ds(REM_BASE, REM_ROWS)])

  return k(x, edge_t)


def _mlp_body(relu_out, x_ref, p_ref, wa_ref, ba_ref, wb_ref, bb_ref, o_ref):
  h = x_ref[...] + p_ref[0] + p_ref[1]
  t = jnp.dot(h, wa_ref[...], preferred_element_type=jnp.float32)
  t = jnp.maximum(t + ba_ref[...], 0.0)
  y = jnp.dot(t, wb_ref[...], preferred_element_type=jnp.float32)
  y = y + bb_ref[...]
  if relu_out:
    y = jnp.maximum(y, 0.0)
  o_ref[...] = y


_ROWS = 1000


def _tc_mlp(x, parts, Wa, ba, Wb, bb, relu_out):
  return pl.pallas_call(
      functools.partial(_mlp_body, relu_out),
      grid=(N_NODES // _ROWS,),
      in_specs=[
          pl.BlockSpec((_ROWS, D), lambda i: (i, 0)),
          pl.BlockSpec((NC, _ROWS, D), lambda i: (0, i, 0)),
          pl.BlockSpec((D, D), lambda i: (0, 0)),
          pl.BlockSpec((1, D), lambda i: (0, 0)),
          pl.BlockSpec((D, D), lambda i: (0, 0)),
          pl.BlockSpec((1, D), lambda i: (0, 0)),
      ],
      out_specs=pl.BlockSpec((_ROWS, D), lambda i: (i, 0)),
      out_shape=jax.ShapeDtypeStruct((N_NODES, D), jnp.float32),
  )(x, parts, Wa, ba.reshape(1, D), Wb, bb.reshape(1, D))


def kernel(x, edge_index, W1, b1, W2, b2, W3, b3, W4, b4):
  src = edge_index[0].astype(jnp.int32)
  dst = edge_index[1].astype(jnp.int32)
  pad = E_PAD - N_EDGES
  src_t = jnp.concatenate([src, jnp.zeros((pad,), jnp.int32)]).reshape(
      NW, NB, BATCH)
  trash_dst = N_NODES + (jnp.arange(pad, dtype=jnp.int32) % TRASH)
  dst_t = jnp.concatenate([dst, trash_dst]).reshape(NW, NB, BATCH)
  edge_t = jnp.stack([src_t, dst_t], axis=2).reshape(
      NW, NCHB, 2 * CHB, BATCH)

  p1 = _sc_aggregate(x, edge_t)
  h = _tc_mlp(x, p1, W1, b1, W2, b2, relu_out=True)
  p2 = _sc_aggregate(h, edge_t)
  return _tc_mlp(h, p2, W3, b3, W4, b4, relu_out=False)


# speedup vs baseline: 1.4430x; 1.4417x over previous
import functools

import jax
import jax.numpy as jnp
from jax import lax
from jax.experimental import pallas as pl
from jax.experimental.pallas import tpu as pltpu
from jax.experimental.pallas import tpu_sc as plsc

N_NODES = 10000
N_EDGES = 320000
D = 128

NC = 2
NS = 16
NW = NC * NS

BATCH = 128
NB = 79
EPT = NB * BATCH
E_PAD = EPT * NW
TRASH = 1024
N_PAD = N_NODES + TRASH
ROWS_PER_SUB = 624
CHUNK = 104
NCHUNK = ROWS_PER_SUB // CHUNK
REM_BASE = ROWS_PER_SUB * NS
REM_ROWS = N_NODES - REM_BASE


def _sc_aggregate(x, edge_t):
  mesh = plsc.VectorSubcoreMesh(core_axis_name="c", subcore_axis_name="s")

  @functools.partial(
      pl.kernel,
      mesh=mesh,
      out_type=jax.ShapeDtypeStruct((NC, N_NODES, D), jnp.float32),
      scratch_types=[
          pltpu.VMEM((2, BATCH), jnp.int32),
          pltpu.VMEM((BATCH, D), jnp.float32),
          pltpu.VMEM((CHUNK, D), jnp.float32),
          pltpu.VMEM_SHARED((N_PAD, D), jnp.float32),
          pltpu.SemaphoreType.DMA,
          pltpu.SemaphoreType.DMA,
      ],
  )
  def k(x_hbm, edge_hbm, out_hbm, eidx, rows, zbuf, aggr, sem, sem2):
    c = lax.axis_index("c")
    s = lax.axis_index("s")
    wid = s * NC + c

    zero = jnp.zeros((16,), jnp.float32)

    def zrow(r, carry):
      for blk in range(D // 16):
        zbuf[r, pl.ds(blk * 16, 16)] = zero
      return carry

    lax.fori_loop(0, CHUNK, zrow, 0)
    r0 = s * ROWS_PER_SUB
    for kk in range(NCHUNK):
      pltpu.sync_copy(zbuf, aggr.at[pl.ds(r0 + kk * CHUNK, CHUNK)])

    @pl.when(s == NS - 1)
    def _zero_rem():
      pltpu.sync_copy(zbuf.at[pl.ds(0, REM_ROWS)],
                      aggr.at[pl.ds(REM_BASE, REM_ROWS)])

    plsc.subcore_barrier()

    def body(j, carry):
      @pl.when(j > 0)
      def _wait_prev_scatter():
        pltpu.make_async_copy(rows, aggr.at[eidx.at[1]], sem2).wait()

      pltpu.sync_copy(edge_hbm.at[wid].at[j], eidx)
      pltpu.async_copy(x_hbm.at[eidx.at[0]], rows, sem).wait()
      pltpu.async_copy(rows, aggr.at[eidx.at[1]], sem2, add=True)
      return carry

    lax.fori_loop(0, NB, body, 0)
    pltpu.make_async_copy(rows, aggr.at[eidx.at[1]], sem2).wait()
    plsc.subcore_barrier()

    for kk in range(NCHUNK):
      pltpu.sync_copy(aggr.at[pl.ds(r0 + kk * CHUNK, CHUNK)], zbuf)
      pltpu.sync_copy(zbuf, out_hbm.at[c].at[pl.ds(r0 + kk * CHUNK, CHUNK)])

    @pl.when(s == NS - 1)
    def _write_rem():
      pltpu.sync_copy(aggr.at[pl.ds(REM_BASE, REM_ROWS)],
                      rows.at[pl.ds(0, REM_ROWS)])
      pltpu.sync_copy(rows.at[pl.ds(0, REM_ROWS)],
                      out_hbm.at[c].at[pl.ds(REM_BASE, REM_ROWS)])

  return k(x, edge_t)


def _mlp_body(relu_out, x_ref, p_ref, wa_ref, ba_ref, wb_ref, bb_ref, o_ref):
  h = x_ref[...] + p_ref[0] + p_ref[1]
  t = jnp.dot(h, wa_ref[...], preferred_element_type=jnp.float32)
  t = jnp.maximum(t + ba_ref[...], 0.0)
  y = jnp.dot(t, wb_ref[...], preferred_element_type=jnp.float32)
  y = y + bb_ref[...]
  if relu_out:
    y = jnp.maximum(y, 0.0)
  o_ref[...] = y


_ROWS = 1000


def _tc_mlp(x, parts, Wa, ba, Wb, bb, relu_out):
  return pl.pallas_call(
      functools.partial(_mlp_body, relu_out),
      grid=(N_NODES // _ROWS,),
      in_specs=[
          pl.BlockSpec((_ROWS, D), lambda i: (i, 0)),
          pl.BlockSpec((NC, _ROWS, D), lambda i: (0, i, 0)),
          pl.BlockSpec((D, D), lambda i: (0, 0)),
          pl.BlockSpec((1, D), lambda i: (0, 0)),
          pl.BlockSpec((D, D), lambda i: (0, 0)),
          pl.BlockSpec((1, D), lambda i: (0, 0)),
      ],
      out_specs=pl.BlockSpec((_ROWS, D), lambda i: (i, 0)),
      out_shape=jax.ShapeDtypeStruct((N_NODES, D), jnp.float32),
  )(x, parts, Wa, ba.reshape(1, D), Wb, bb.reshape(1, D))


def kernel(x, edge_index, W1, b1, W2, b2, W3, b3, W4, b4):
  src = edge_index[0].astype(jnp.int32)
  dst = edge_index[1].astype(jnp.int32)
  pad = E_PAD - N_EDGES
  src_t = jnp.concatenate([src, jnp.zeros((pad,), jnp.int32)]).reshape(
      NW, NB, BATCH)
  trash_dst = N_NODES + (jnp.arange(pad, dtype=jnp.int32) % TRASH)
  dst_t = jnp.concatenate([dst, trash_dst]).reshape(NW, NB, BATCH)
  edge_t = jnp.stack([src_t, dst_t], axis=2)

  p1 = _sc_aggregate(x, edge_t)
  h = _tc_mlp(x, p1, W1, b1, W2, b2, relu_out=True)
  p2 = _sc_aggregate(h, edge_t)
  return _tc_mlp(h, p2, W3, b3, W4, b4, relu_out=False)


# speedup vs baseline: 1.6090x; 1.1151x over previous
import functools

import jax
import jax.numpy as jnp
from jax import lax
from jax.experimental import pallas as pl
from jax.experimental.pallas import tpu as pltpu
from jax.experimental.pallas import tpu_sc as plsc

N_NODES = 10000
N_EDGES = 320000
D = 128

NC = 2
NS = 16
NW = NC * NS

BATCH = 128
NB = 79
CHB = 4
NCHB = NB // CHB
NTAIL = NB - NCHB * CHB
EPT = NB * BATCH
E_PAD = EPT * NW
TRASH = 1024
N_PAD = N_NODES + TRASH
ROWS_PER_SUB = 624
CHUNK = 104
NCHUNK = ROWS_PER_SUB // CHUNK
REM_BASE = ROWS_PER_SUB * NS
REM_ROWS = N_NODES - REM_BASE


def _sc_aggregate(x, edge_t, tail_t):
  mesh = plsc.VectorSubcoreMesh(core_axis_name="c", subcore_axis_name="s")

  @functools.partial(
      pl.kernel,
      mesh=mesh,
      out_type=jax.ShapeDtypeStruct((NC, N_NODES, D), jnp.float32),
      scratch_types=[
          pltpu.VMEM((2 * CHB, BATCH), jnp.int32),
          pltpu.VMEM((BATCH, D), jnp.float32),
          pltpu.VMEM((CHUNK, D), jnp.float32),
          pltpu.VMEM_SHARED((N_PAD, D), jnp.float32),
          pltpu.SemaphoreType.DMA,
          pltpu.SemaphoreType.DMA,
      ],
  )
  def k(x_hbm, edge_hbm, tail_hbm, out_hbm, eidx, rows, zbuf, aggr, sem,
        sem2):
    c = lax.axis_index("c")
    s = lax.axis_index("s")
    wid = s * NC + c

    zero = jnp.zeros((16,), jnp.float32)

    def zrow(r, carry):
      for blk in range(D // 16):
        zbuf[r, pl.ds(blk * 16, 16)] = zero
      return carry

    lax.fori_loop(0, CHUNK, zrow, 0)
    r0 = s * ROWS_PER_SUB
    for kk in range(NCHUNK):
      pltpu.sync_copy(zbuf, aggr.at[pl.ds(r0 + kk * CHUNK, CHUNK)])

    @pl.when(s == NS - 1)
    def _zero_rem():
      pltpu.sync_copy(zbuf.at[pl.ds(0, REM_ROWS)],
                      aggr.at[pl.ds(REM_BASE, REM_ROWS)])

    plsc.subcore_barrier()

    def body(ci, carry):
      @pl.when(ci > 0)
      def _wait_prev_scatter():
        pltpu.make_async_copy(rows, aggr.at[eidx.at[1]], sem2).wait()

      pltpu.sync_copy(edge_hbm.at[wid].at[ci], eidx)
      for k in range(CHB):
        if k > 0:
          pltpu.make_async_copy(rows, aggr.at[eidx.at[2 * k - 1]],
                                sem2).wait()
        pltpu.async_copy(x_hbm.at[eidx.at[2 * k]], rows, sem).wait()
        pltpu.async_copy(rows, aggr.at[eidx.at[2 * k + 1]], sem2, add=True)
      return carry

    lax.fori_loop(0, NCHB, body, 0)

    pltpu.make_async_copy(rows, aggr.at[eidx.at[1]], sem2).wait()
    pltpu.sync_copy(tail_hbm.at[wid], eidx.at[pl.ds(0, 2 * NTAIL)])
    for k in range(NTAIL):
      if k > 0:
        pltpu.make_async_copy(rows, aggr.at[eidx.at[2 * k - 1]], sem2).wait()
      pltpu.async_copy(x_hbm.at[eidx.at[2 * k]], rows, sem).wait()
      pltpu.async_copy(rows, aggr.at[eidx.at[2 * k + 1]], sem2, add=True)
    pltpu.make_async_copy(rows, aggr.at[eidx.at[1]], sem2).wait()
    plsc.subcore_barrier()

    for kk in range(NCHUNK):
      pltpu.sync_copy(aggr.at[pl.ds(r0 + kk * CHUNK, CHUNK)], zbuf)
      pltpu.sync_copy(zbuf, out_hbm.at[c].at[pl.ds(r0 + kk * CHUNK, CHUNK)])

    @pl.when(s == NS - 1)
    def _write_rem():
      pltpu.sync_copy(aggr.at[pl.ds(REM_BASE, REM_ROWS)],
                      rows.at[pl.ds(0, REM_ROWS)])
      pltpu.sync_copy(rows.at[pl.ds(0, REM_ROWS)],
                      out_hbm.at[c].at[pl.ds(REM_BASE, REM_ROWS)])

  return k(x, edge_t, tail_t)


def _mlp_body(relu_out, x_ref, p_ref, wa_ref, ba_ref, wb_ref, bb_ref, o_ref):
  h = x_ref[...] + p_ref[0] + p_ref[1]
  t = jnp.dot(h, wa_ref[...], preferred_element_type=jnp.float32)
  t = jnp.maximum(t + ba_ref[...], 0.0)
  y = jnp.dot(t, wb_ref[...], preferred_element_type=jnp.float32)
  y = y + bb_ref[...]
  if relu_out:
    y = jnp.maximum(y, 0.0)
  o_ref[...] = y


_ROWS = 1000


def _tc_mlp(x, parts, Wa, ba, Wb, bb, relu_out):
  return pl.pallas_call(
      functools.partial(_mlp_body, relu_out),
      grid=(N_NODES // _ROWS,),
      in_specs=[
          pl.BlockSpec((_ROWS, D), lambda i: (i, 0)),
          pl.BlockSpec((NC, _ROWS, D), lambda i: (0, i, 0)),
          pl.BlockSpec((D, D), lambda i: (0, 0)),
          pl.BlockSpec((1, D), lambda i: (0, 0)),
          pl.BlockSpec((D, D), lambda i: (0, 0)),
          pl.BlockSpec((1, D), lambda i: (0, 0)),
      ],
      out_specs=pl.BlockSpec((_ROWS, D), lambda i: (i, 0)),
      out_shape=jax.ShapeDtypeStruct((N_NODES, D), jnp.float32),
  )(x, parts, Wa, ba.reshape(1, D), Wb, bb.reshape(1, D))


def kernel(x, edge_index, W1, b1, W2, b2, W3, b3, W4, b4):
  src = edge_index[0].astype(jnp.int32)
  dst = edge_index[1].astype(jnp.int32)
  pad = E_PAD - N_EDGES
  src_t = jnp.concatenate([src, jnp.zeros((pad,), jnp.int32)]).reshape(
      NW, NB, BATCH)
  trash_dst = N_NODES + (jnp.arange(pad, dtype=jnp.int32) % TRASH)
  dst_t = jnp.concatenate([dst, trash_dst]).reshape(NW, NB, BATCH)
  ei = jnp.stack([src_t, dst_t], axis=2)
  edge_t = ei[:, :NCHB * CHB].reshape(NW, NCHB, 2 * CHB, BATCH)
  tail_t = ei[:, NCHB * CHB:].reshape(NW, 2 * NTAIL, BATCH)

  p1 = _sc_aggregate(x, edge_t, tail_t)
  h = _tc_mlp(x, p1, W1, b1, W2, b2, relu_out=True)
  p2 = _sc_aggregate(h, edge_t, tail_t)
  return _tc_mlp(h, p2, W3, b3, W4, b4, relu_out=False)


# speedup vs baseline: 1.7009x; 1.0571x over previous
import functools

import jax
import jax.numpy as jnp
from jax import lax
from jax.experimental import pallas as pl
from jax.experimental.pallas import tpu as pltpu
from jax.experimental.pallas import tpu_sc as plsc

N_NODES = 10000
N_EDGES = 320000
D = 128

NC = 2
NS = 16
NW = NC * NS

BATCH = 128
NB = 79
CHB = 4
NCHB = NB // CHB
NTAIL = NB - NCHB * CHB
EPT = NB * BATCH
E_PAD = EPT * NW
TRASH = 128
N_PAD = N_NODES + TRASH
ROWS_PER_SUB = 624
CHUNK = 48
NCHUNK = ROWS_PER_SUB // CHUNK
REM_BASE = ROWS_PER_SUB * NS
REM_ROWS = N_NODES - REM_BASE


def _sc_aggregate(x, edge_t, tail_t):
  mesh = plsc.VectorSubcoreMesh(core_axis_name="c", subcore_axis_name="s")

  @functools.partial(
      pl.kernel,
      mesh=mesh,
      out_type=jax.ShapeDtypeStruct((NC, N_NODES, D), jnp.float32),
      scratch_types=[
          pltpu.VMEM((2 * CHB, BATCH), jnp.int32),
          pltpu.VMEM((BATCH, D), jnp.float32),
          pltpu.VMEM((CHUNK, D), jnp.float32),
          pltpu.VMEM_SHARED((N_PAD, D), jnp.float32),
          pltpu.SemaphoreType.DMA,
          pltpu.SemaphoreType.DMA,
          pltpu.VMEM((BATCH, D), jnp.float32),
          pltpu.SemaphoreType.DMA,
          pltpu.SemaphoreType.DMA,
      ],
  )
  def k(x_hbm, edge_hbm, tail_hbm, out_hbm, eidx, rows, zbuf, aggr, sga,
        ssa, rows2, sgb, ssb):
    c = lax.axis_index("c")
    s = lax.axis_index("s")
    wid = s * NC + c

    zero = jnp.zeros((16,), jnp.float32)

    def zrow(r, carry):
      for blk in range(D // 16):
        zbuf[r, pl.ds(blk * 16, 16)] = zero
      return carry

    lax.fori_loop(0, CHUNK, zrow, 0)
    r0 = s * ROWS_PER_SUB
    for kk in range(NCHUNK):
      pltpu.sync_copy(zbuf, aggr.at[pl.ds(r0 + kk * CHUNK, CHUNK)])

    @pl.when(s == NS - 1)
    def _zero_rem():
      pltpu.sync_copy(zbuf.at[pl.ds(0, REM_ROWS)],
                      aggr.at[pl.ds(REM_BASE, REM_ROWS)])

    plsc.subcore_barrier()

    def wait_ga():
      pltpu.make_async_copy(x_hbm.at[eidx.at[0]], rows, sga).wait()

    def wait_gb():
      pltpu.make_async_copy(x_hbm.at[eidx.at[0]], rows2, sgb).wait()

    def wait_sa():
      pltpu.make_async_copy(rows, aggr.at[eidx.at[1]], ssa).wait()

    def wait_sb():
      pltpu.make_async_copy(rows2, aggr.at[eidx.at[1]], ssb).wait()

    def body(ci, carry):
      @pl.when(ci > 0)
      def _wait_prev():
        wait_sa()
        wait_sb()

      pltpu.sync_copy(edge_hbm.at[wid].at[ci], eidx)
      pltpu.async_copy(x_hbm.at[eidx.at[0]], rows, sga)
      pltpu.async_copy(x_hbm.at[eidx.at[2]], rows2, sgb)
      wait_ga()
      pltpu.async_copy(rows, aggr.at[eidx.at[1]], ssa, add=True)
      wait_gb()
      pltpu.async_copy(rows2, aggr.at[eidx.at[3]], ssb, add=True)
      wait_sa()
      pltpu.async_copy(x_hbm.at[eidx.at[4]], rows, sga)
      wait_sb()
      pltpu.async_copy(x_hbm.at[eidx.at[6]], rows2, sgb)
      wait_ga()
      pltpu.async_copy(rows, aggr.at[eidx.at[5]], ssa, add=True)
      wait_gb()
      pltpu.async_copy(rows2, aggr.at[eidx.at[7]], ssb, add=True)
      return carry

    lax.fori_loop(0, NCHB, body, 0)

    wait_sa()
    wait_sb()
    pltpu.sync_copy(tail_hbm.at[wid], eidx.at[pl.ds(0, 2 * NTAIL)])
    pltpu.async_copy(x_hbm.at[eidx.at[0]], rows, sga)
    pltpu.async_copy(x_hbm.at[eidx.at[2]], rows2, sgb)
    wait_ga()
    pltpu.async_copy(rows, aggr.at[eidx.at[1]], ssa, add=True)
    wait_gb()
    pltpu.async_copy(rows2, aggr.at[eidx.at[3]], ssb, add=True)
    wait_sa()
    pltpu.async_copy(x_hbm.at[eidx.at[4]], rows, sga)
    wait_ga()
    pltpu.async_copy(rows, aggr.at[eidx.at[5]], ssa, add=True)
    wait_sa()
    wait_sb()
    plsc.subcore_barrier()

    for kk in range(NCHUNK):
      pltpu.sync_copy(aggr.at[pl.ds(r0 + kk * CHUNK, CHUNK)], zbuf)
      pltpu.sync_copy(zbuf, out_hbm.at[c].at[pl.ds(r0 + kk * CHUNK, CHUNK)])

    @pl.when(s == NS - 1)
    def _write_rem():
      pltpu.sync_copy(aggr.at[pl.ds(REM_BASE, REM_ROWS)],
                      rows.at[pl.ds(0, REM_ROWS)])
      pltpu.sync_copy(rows.at[pl.ds(0, REM_ROWS)],
                      out_hbm.at[c].at[pl.ds(REM_BASE, REM_ROWS)])

  return k(x, edge_t, tail_t)


def _mlp_body(relu_out, x_ref, p_ref, wa_ref, ba_ref, wb_ref, bb_ref, o_ref):
  h = x_ref[...] + p_ref[0] + p_ref[1]
  t = jnp.dot(h, wa_ref[...], preferred_element_type=jnp.float32)
  t = jnp.maximum(t + ba_ref[...], 0.0)
  y = jnp.dot(t, wb_ref[...], preferred_element_type=jnp.float32)
  y = y + bb_ref[...]
  if relu_out:
    y = jnp.maximum(y, 0.0)
  o_ref[...] = y


_ROWS = 1000


def _tc_mlp(x, parts, Wa, ba, Wb, bb, relu_out):
  return pl.pallas_call(
      functools.partial(_mlp_body, relu_out),
      grid=(N_NODES // _ROWS,),
      in_specs=[
          pl.BlockSpec((_ROWS, D), lambda i: (i, 0)),
          pl.BlockSpec((NC, _ROWS, D), lambda i: (0, i, 0)),
          pl.BlockSpec((D, D), lambda i: (0, 0)),
          pl.BlockSpec((1, D), lambda i: (0, 0)),
          pl.BlockSpec((D, D), lambda i: (0, 0)),
          pl.BlockSpec((1, D), lambda i: (0, 0)),
      ],
      out_specs=pl.BlockSpec((_ROWS, D), lambda i: (i, 0)),
      out_shape=jax.ShapeDtypeStruct((N_NODES, D), jnp.float32),
  )(x, parts, Wa, ba.reshape(1, D), Wb, bb.reshape(1, D))


def kernel(x, edge_index, W1, b1, W2, b2, W3, b3, W4, b4):
  src = edge_index[0].astype(jnp.int32)
  dst = edge_index[1].astype(jnp.int32)
  pad = E_PAD - N_EDGES
  src_t = jnp.concatenate([src, jnp.zeros((pad,), jnp.int32)]).reshape(
      NW, NB, BATCH)
  trash_dst = N_NODES + (jnp.arange(pad, dtype=jnp.int32) % TRASH)
  dst_t = jnp.concatenate([dst, trash_dst]).reshape(NW, NB, BATCH)
  ei = jnp.stack([src_t, dst_t], axis=2)
  edge_t = ei[:, :NCHB * CHB].reshape(NW, NCHB, 2 * CHB, BATCH)
  tail_t = ei[:, NCHB * CHB:].reshape(NW, 2 * NTAIL, BATCH)

  p1 = _sc_aggregate(x, edge_t, tail_t)
  h = _tc_mlp(x, p1, W1, b1, W2, b2, relu_out=True)
  p2 = _sc_aggregate(h, edge_t, tail_t)
  return _tc_mlp(h, p2, W3, b3, W4, b4, relu_out=False)
